# Initial kernel scaffold; baseline (speedup 1.0000x reference)
import jax
import jax.numpy as jnp
from jax.experimental import pallas as pl


def kernel(feat, pos, edge_index, W0m, W0s, b0, R0a, r0a, R0b, r0b, Wmm, Wms, bm, Rma, rma, Rmb, rmb, W1m, W1s, b1, R1a, r1a, R1b, r1b):
    raise NotImplementedError("write your pallas kernel here")


# speedup vs baseline: 3.3432x; 3.3432x over previous
import functools

import jax
import jax.numpy as jnp
from jax import lax
from jax.experimental import pallas as pl
from jax.experimental.pallas import tpu as pltpu
from jax.experimental.pallas import tpu_sc as plsc

_N = 10000
_E = 320000
_D = 128
_H = 32
_NC = 2
_NS = 16
_NW = _NC * _NS
_L = 16
_W = 128
_NWIN = _E // _W
_WIN_BASE = _NWIN // _NW
_WIN_REM = _NWIN % _NW
_RPT = 624
_RTAIL = _N - _NS * _RPT


def _sc_mesh():
    return plsc.VectorSubcoreMesh(
        core_axis_name="c", subcore_axis_name="s",
        num_cores=_NC, num_subcores=_NS)


_SC_PARAMS = pltpu.CompilerParams(needs_layout_passes=False)


def _worker_id():
    return lax.axis_index("s") * _NC + lax.axis_index("c")


def _num_windows(wid):
    return _WIN_BASE + jnp.where(wid < _WIN_REM, 1, 0).astype(jnp.int32)


def _edge_dist2(px, py, pz, src, dst):
    @functools.partial(
        pl.kernel,
        out_type=jax.ShapeDtypeStruct((_E,), jnp.float32),
        mesh=_sc_mesh(),
        compiler_params=_SC_PARAMS,
        scratch_types=[
            pltpu.VMEM((_N,), jnp.float32),
            pltpu.VMEM((_N,), jnp.float32),
            pltpu.VMEM((_N,), jnp.float32),
            pltpu.VMEM((_W,), jnp.int32),
            pltpu.VMEM((_W,), jnp.int32),
            pltpu.VMEM((_W,), jnp.float32),
        ],
    )
    def k(px_h, py_h, pz_h, src_h, dst_h, d2_h, px_v, py_v, pz_v,
          src_v, dst_v, d2_v):
        wid = _worker_id()
        pltpu.sync_copy(px_h, px_v)
        pltpu.sync_copy(py_h, py_v)
        pltpu.sync_copy(pz_h, pz_v)

        def win(i, carry):
            base = (wid + _NW * i) * _W
            pltpu.sync_copy(src_h.at[pl.ds(base, _W)], src_v)
            pltpu.sync_copy(dst_h.at[pl.ds(base, _W)], dst_v)

            def blk(kk, c2):
                sl = pl.ds(kk * _L, _L)
                sv = src_v[sl]
                dv = dst_v[sl]
                dx = plsc.load_gather(px_v, [sv]) - plsc.load_gather(px_v, [dv])
                dy = plsc.load_gather(py_v, [sv]) - plsc.load_gather(py_v, [dv])
                dz = plsc.load_gather(pz_v, [sv]) - plsc.load_gather(pz_v, [dv])
                d2_v[sl] = dx * dx + dy * dy + dz * dz
                return c2

            lax.fori_loop(0, _W // _L, blk, 0)
            pltpu.sync_copy(d2_v, d2_h.at[pl.ds(base, _W)])
            return carry

        lax.fori_loop(0, _num_windows(wid), win, 0)

    return k(px, py, pz, src, dst)


def _conv_edges(m, gate, src, dst, zeros_nd):
    @functools.partial(
        pl.kernel,
        out_type=jax.ShapeDtypeStruct((_NC * _N, _D), jnp.float32),
        mesh=_sc_mesh(),
        compiler_params=_SC_PARAMS,
        scratch_types=[
            pltpu.VMEM_SHARED((_N, _D), jnp.float32),
            pltpu.VMEM((_W,), jnp.int32),
            pltpu.VMEM((_W,), jnp.int32),
            pltpu.VMEM((_W, _D), jnp.float32),
            pltpu.VMEM((_W, _D), jnp.float32),
            pltpu.SemaphoreType.DMA,
        ],
    )
    def k(m_h, g_h, src_h, dst_h, z_h, out_h, agg_s, src_v, dst_v,
          rows_v, gate_v, sem):
        cid = lax.axis_index("c")
        sid = lax.axis_index("s")
        wid = _worker_id()
        pltpu.sync_copy(z_h.at[pl.ds(sid * _RPT, _RPT)],
                        agg_s.at[pl.ds(sid * _RPT, _RPT)])
        @pl.when(sid == 0)
        def _zero_tail():
            pltpu.sync_copy(z_h.at[pl.ds(_NS * _RPT, _RTAIL)],
                            agg_s.at[pl.ds(_NS * _RPT, _RTAIL)])
        plsc.subcore_barrier()

        def win(i, carry):
            base = (wid + _NW * i) * _W
            pltpu.sync_copy(src_h.at[pl.ds(base, _W)], src_v)
            pltpu.sync_copy(dst_h.at[pl.ds(base, _W)], dst_v)
            pltpu.async_copy(m_h.at[src_v], rows_v, sem).wait()
            pltpu.sync_copy(g_h.at[pl.ds(base, _W)], gate_v)

            def row(r, c2):
                for j in range(_D // _L):
                    sl = pl.ds(j * _L, _L)
                    rows_v[r, sl] = rows_v[r, sl] * gate_v[r, sl]
                return c2

            lax.fori_loop(0, _W, row, 0)
            pltpu.sync_copy(rows_v, agg_s.at[dst_v], add=True)
            return carry

        lax.fori_loop(0, _num_windows(wid), win, 0)
        plsc.subcore_barrier()
        pltpu.sync_copy(
            agg_s.at[pl.ds(sid * _RPT, _RPT)],
            out_h.at[pl.ds(cid * _N + sid * _RPT, _RPT)])
        @pl.when(sid == 0)
        def _out_tail():
            pltpu.sync_copy(
                agg_s.at[pl.ds(_NS * _RPT, _RTAIL)],
                out_h.at[pl.ds(cid * _N + _NS * _RPT, _RTAIL)])

    return k(m, gate, src, dst, zeros_nd)


def _ln(x):
    x = x - jnp.mean(x, axis=-1, keepdims=True)
    return x / jnp.sqrt(jnp.mean(x * x, axis=-1, keepdims=True) + 1e-5)


def _dot(a, b):
    return jnp.dot(a, b, preferred_element_type=jnp.float32)


_EB = 1280
_EGRID = _E // _EB


def _gates(d2, R0a, r0a, R0b, r0b, Rma, rma, Rmb, rmb, R1a, r1a, R1b, r1b):
    d2r = d2.reshape(_E, 1)

    def body(d2_ref, a0, c0, b0_, d0, am, cm, bm_, dm, a1, c1, b1_, d1,
             g0_o, gm_o, g1_o):
        dcol = jnp.sqrt(d2_ref[...] + 1e-12)

        def one(Ra, ra, Rb, rb):
            h = jnp.maximum(dcol * Ra[...] + ra[...], 0.0)
            return _dot(h, Rb[...]) + rb[...]

        g0_o[...] = one(a0, c0, b0_, d0)
        gm_o[...] = one(am, cm, bm_, dm)
        g1_o[...] = one(a1, c1, b1_, d1)

    w = [R0a, r0a.reshape(1, _H), R0b, r0b.reshape(1, _D),
         Rma, rma.reshape(1, _H), Rmb, rmb.reshape(1, _D),
         R1a, r1a.reshape(1, _H), R1b, r1b.reshape(1, _D)]
    wspecs = [pl.BlockSpec(x.shape, lambda i: (0,) * x.ndim) for x in w]
    gs = jax.ShapeDtypeStruct((_E, _D), jnp.float32)
    return pl.pallas_call(
        body,
        grid=(_EGRID,),
        in_specs=[pl.BlockSpec((_EB, 1), lambda i: (i, 0))] + wspecs,
        out_specs=[pl.BlockSpec((_EB, _D), lambda i: (i, 0))] * 3,
        out_shape=(gs, gs, gs),
    )(d2r, *w)


def _pre(feat, W0m, W0s, b0):
    def body(f, wm, ws, b, m_o, s_o):
        xn = _ln(f[...])
        m_o[...] = _dot(xn, wm[...])
        s_o[...] = _dot(xn, ws[...]) + b[...]

    nd = jax.ShapeDtypeStruct((_N, _D), jnp.float32)
    return pl.pallas_call(body, out_shape=(nd, nd))(
        feat, W0m, W0s, b0.reshape(1, _D))


def _combine(aggp, s, xskip, wm, ws, b):
    has_skip = xskip is not None

    def body(ag, s_r, *refs):
        if has_skip:
            xs_r, wm_r, ws_r, b_r, y_o, m_o, s_o = refs
        else:
            wm_r, ws_r, b_r, y_o, m_o, s_o = refs
        y = jnp.maximum(ag[0:_N, :] + ag[_N:2 * _N, :] + s_r[...], 0.0)
        if has_skip:
            y = y + xs_r[...]
        y_o[...] = y
        xn = _ln(y)
        m_o[...] = _dot(xn, wm_r[...])
        s_o[...] = _dot(xn, ws_r[...]) + b_r[...]

    nd = jax.ShapeDtypeStruct((_N, _D), jnp.float32)
    args = (aggp, s) + ((xskip,) if has_skip else ()) + (
        wm, ws, b.reshape(1, _D))
    return pl.pallas_call(body, out_shape=(nd, nd, nd))(*args)


def _final(aggp, s):
    def body(ag, s_r, out_o):
        y = jnp.maximum(ag[0:_N, :] + ag[_N:2 * _N, :] + s_r[...], 0.0)
        out_o[...] = _ln(y)

    return pl.pallas_call(
        body, out_shape=jax.ShapeDtypeStruct((_N, _D), jnp.float32))(aggp, s)


def kernel(feat, pos, edge_index, W0m, W0s, b0, R0a, r0a, R0b, r0b,
           Wmm, Wms, bm, Rma, rma, Rmb, rmb, W1m, W1s, b1, R1a, r1a,
           R1b, r1b):
    src = edge_index[0]
    dst = edge_index[1]
    px = pos[:, 0]
    py = pos[:, 1]
    pz = pos[:, 2]

    d2 = _edge_dist2(px, py, pz, src, dst)
    g0, gm, g1 = _gates(d2, R0a, r0a, R0b, r0b, Rma, rma, Rmb, rmb,
                        R1a, r1a, R1b, r1b)
    z = jnp.zeros((_N, _D), jnp.float32)

    m, s = _pre(feat, W0m, W0s, b0)
    aggp = _conv_edges(m, g0, src, dst, z)
    x, m, s = _combine(aggp, s, None, Wmm, Wms, bm)
    for i in range(3):
        aggp = _conv_edges(m, gm, src, dst, z)
        if i < 2:
            x, m, s = _combine(aggp, s, x, Wmm, Wms, bm)
        else:
            x, m, s = _combine(aggp, s, x, W1m, W1s, b1)
    aggp = _conv_edges(m, g1, src, dst, z)
    return _final(aggp, s)


# speedup vs baseline: 5.4620x; 1.6337x over previous
import functools

import jax
import jax.numpy as jnp
from jax import lax
from jax.experimental import pallas as pl
from jax.experimental.pallas import tpu as pltpu
from jax.experimental.pallas import tpu_sc as plsc

_N = 10000
_E = 320000
_D = 128
_H = 32
_NC = 2
_NS = 16
_NW = _NC * _NS
_L = 16
_W = 64
_NWIN = _E // _W
_WIN_BASE = _NWIN // _NW
_WIN_REM = _NWIN % _NW
_RPT = 624
_RTAIL = _N - _NS * _RPT


def _sc_mesh():
    return plsc.VectorSubcoreMesh(
        core_axis_name="c", subcore_axis_name="s",
        num_cores=_NC, num_subcores=_NS)


_SC_PARAMS = pltpu.CompilerParams(needs_layout_passes=False)


def _worker_id():
    return lax.axis_index("s") * _NC + lax.axis_index("c")


def _num_windows(wid):
    return _WIN_BASE + jnp.where(wid < _WIN_REM, 1, 0).astype(jnp.int32)


def _edge_dist2(px, py, pz, src, dst):
    @functools.partial(
        pl.kernel,
        out_type=jax.ShapeDtypeStruct((_E,), jnp.float32),
        mesh=_sc_mesh(),
        compiler_params=_SC_PARAMS,
        scratch_types=[
            pltpu.VMEM((_N,), jnp.float32),
            pltpu.VMEM((_N,), jnp.float32),
            pltpu.VMEM((_N,), jnp.float32),
            pltpu.VMEM((_W,), jnp.int32),
            pltpu.VMEM((_W,), jnp.int32),
            pltpu.VMEM((_W,), jnp.float32),
        ],
    )
    def k(px_h, py_h, pz_h, src_h, dst_h, d2_h, px_v, py_v, pz_v,
          src_v, dst_v, d2_v):
        wid = _worker_id()
        pltpu.sync_copy(px_h, px_v)
        pltpu.sync_copy(py_h, py_v)
        pltpu.sync_copy(pz_h, pz_v)

        def win(i, carry):
            base = (wid + _NW * i) * _W
            pltpu.sync_copy(src_h.at[pl.ds(base, _W)], src_v)
            pltpu.sync_copy(dst_h.at[pl.ds(base, _W)], dst_v)

            def blk(kk, c2):
                sl = pl.ds(kk * _L, _L)
                sv = src_v[sl]
                dv = dst_v[sl]
                dx = plsc.load_gather(px_v, [sv]) - plsc.load_gather(px_v, [dv])
                dy = plsc.load_gather(py_v, [sv]) - plsc.load_gather(py_v, [dv])
                dz = plsc.load_gather(pz_v, [sv]) - plsc.load_gather(pz_v, [dv])
                d2_v[sl] = dx * dx + dy * dy + dz * dz
                return c2

            lax.fori_loop(0, _W // _L, blk, 0)
            pltpu.sync_copy(d2_v, d2_h.at[pl.ds(base, _W)])
            return carry

        lax.fori_loop(0, _num_windows(wid), win, 0)

    return k(px, py, pz, src, dst)


_TRIP = 160


def _conv_edges(m, gate, src, dst, zeros_nd):
    @functools.partial(
        pl.kernel,
        out_type=jax.ShapeDtypeStruct((_NC * _N, _D), jnp.float32),
        mesh=_sc_mesh(),
        compiler_params=_SC_PARAMS,
        scratch_types=(
            [pltpu.VMEM_SHARED((_N, _D), jnp.float32)]
            + [pltpu.VMEM((_W,), jnp.int32)] * 8
            + [pltpu.VMEM((_W, _D), jnp.float32)] * 4
            + [pltpu.SemaphoreType.DMA] * 10
        ),
    )
    def k(m_h, g_h, src_h, dst_h, z_h, out_h, agg_s,
          sb0, sb1, sb2, sb3, db0, db1, db2, db3, r0, r1, gb0, gb1,
          gs0, gs1, gts0, gts1, ss0, ss1, is0, is1, is2, is3):
        srcb = [sb0, sb1, sb2, sb3]
        dstb = [db0, db1, db2, db3]
        rows = [r0, r1]
        gateb = [gb0, gb1]
        gsem = [gs0, gs1]
        gtsem = [gts0, gts1]
        ssem = [ss0, ss1]
        isem = [is0, is1, is2, is3]

        cid = lax.axis_index("c")
        sid = lax.axis_index("s")
        wid = _worker_id()
        n = _num_windows(wid)

        def base_of(w):
            return (wid + _NW * jnp.minimum(w, n - 1)) * _W

        def issue_idx(w, j):
            b = base_of(w)
            pltpu.async_copy(src_h.at[pl.ds(b, _W)], srcb[j], isem[j])
            pltpu.async_copy(dst_h.at[pl.ds(b, _W)], dstb[j], isem[j])

        def wait_idx(j):
            pltpu.make_async_copy(
                src_h.at[pl.ds(0, _W)], srcb[j], isem[j]).wait()
            pltpu.make_async_copy(
                dst_h.at[pl.ds(0, _W)], dstb[j], isem[j]).wait()

        def issue_data(w, j, p):
            pltpu.async_copy(m_h.at[srcb[j]], rows[p], gsem[p])
            pltpu.async_copy(g_h.at[pl.ds(base_of(w), _W)], gateb[p],
                             gtsem[p])

        def wait_data(j, p):
            pltpu.make_async_copy(m_h.at[srcb[j]], rows[p], gsem[p]).wait()
            pltpu.make_async_copy(
                g_h.at[pl.ds(0, _W)], gateb[p], gtsem[p]).wait()

        def issue_scatter(j, p):
            pltpu.async_copy(rows[p], agg_s.at[dstb[j]], ssem[p], add=True)

        def wait_scatter(j, p):
            pltpu.make_async_copy(rows[p], agg_s.at[dstb[j]], ssem[p]).wait()

        pltpu.sync_copy(z_h.at[pl.ds(sid * _RPT, _RPT)],
                        agg_s.at[pl.ds(sid * _RPT, _RPT)])
        @pl.when(sid == 0)
        def _zero_tail():
            pltpu.sync_copy(z_h.at[pl.ds(_NS * _RPT, _RTAIL)],
                            agg_s.at[pl.ds(_NS * _RPT, _RTAIL)])
        plsc.subcore_barrier()

        pltpu.sync_copy(src_h.at[pl.ds(base_of(0), _W)], srcb[0])
        pltpu.sync_copy(dst_h.at[pl.ds(base_of(0), _W)], dstb[0])
        issue_data(0, 0, 0)
        issue_idx(1, 1)

        @pl.loop(0, _TRIP, step=4)
        def outer(i):
            for b in range(4):
                w = i + b
                p = b % 2
                q = (b + 1) % 2

                @pl.when(jnp.logical_and(w >= 1, w - 1 < n))
                def _done_prev():
                    wait_scatter((b + 3) % 4, q)

                wait_idx((b + 1) % 4)
                issue_data(w + 1, (b + 1) % 4, q)
                issue_idx(w + 2, (b + 2) % 4)
                wait_data(b, p)

                @plsc.parallel_loop(0, _W, unroll=2)
                def _mul(r):
                    for jj in range(_D // _L):
                        sl = pl.ds(jj * _L, _L)
                        rows[p][r, sl] = rows[p][r, sl] * gateb[p][r, sl]

                @pl.when(w < n)
                def _scatter():
                    issue_scatter(b, p)

        wait_data(0, 0)
        wait_idx(1)
        plsc.subcore_barrier()
        pltpu.sync_copy(
            agg_s.at[pl.ds(sid * _RPT, _RPT)],
            out_h.at[pl.ds(cid * _N + sid * _RPT, _RPT)])
        @pl.when(sid == 0)
        def _out_tail():
            pltpu.sync_copy(
                agg_s.at[pl.ds(_NS * _RPT, _RTAIL)],
                out_h.at[pl.ds(cid * _N + _NS * _RPT, _RTAIL)])

    return k(m, gate, src, dst, zeros_nd)


def _ln(x):
    x = x - jnp.mean(x, axis=-1, keepdims=True)
    return x / jnp.sqrt(jnp.mean(x * x, axis=-1, keepdims=True) + 1e-5)


def _dot(a, b):
    return jnp.dot(a, b, preferred_element_type=jnp.float32)


_EB = 1280
_EGRID = _E // _EB


def _gates(d2, R0a, r0a, R0b, r0b, Rma, rma, Rmb, rmb, R1a, r1a, R1b, r1b):
    d2r = d2.reshape(_E, 1)

    def body(d2_ref, a0, c0, b0_, d0, am, cm, bm_, dm, a1, c1, b1_, d1,
             g0_o, gm_o, g1_o):
        dcol = jnp.sqrt(d2_ref[...] + 1e-12)

        def one(Ra, ra, Rb, rb):
            h = jnp.maximum(dcol * Ra[...] + ra[...], 0.0)
            return _dot(h, Rb[...]) + rb[...]

        g0_o[...] = one(a0, c0, b0_, d0)
        gm_o[...] = one(am, cm, bm_, dm)
        g1_o[...] = one(a1, c1, b1_, d1)

    w = [R0a, r0a.reshape(1, _H), R0b, r0b.reshape(1, _D),
         Rma, rma.reshape(1, _H), Rmb, rmb.reshape(1, _D),
         R1a, r1a.reshape(1, _H), R1b, r1b.reshape(1, _D)]
    wspecs = [pl.BlockSpec(x.shape, lambda i: (0,) * x.ndim) for x in w]
    gs = jax.ShapeDtypeStruct((_E, _D), jnp.float32)
    return pl.pallas_call(
        body,
        grid=(_EGRID,),
        in_specs=[pl.BlockSpec((_EB, 1), lambda i: (i, 0))] + wspecs,
        out_specs=[pl.BlockSpec((_EB, _D), lambda i: (i, 0))] * 3,
        out_shape=(gs, gs, gs),
    )(d2r, *w)


def _pre(feat, W0m, W0s, b0):
    def body(f, wm, ws, b, m_o, s_o):
        xn = _ln(f[...])
        m_o[...] = _dot(xn, wm[...])
        s_o[...] = _dot(xn, ws[...]) + b[...]

    nd = jax.ShapeDtypeStruct((_N, _D), jnp.float32)
    return pl.pallas_call(body, out_shape=(nd, nd))(
        feat, W0m, W0s, b0.reshape(1, _D))


def _combine(aggp, s, xskip, wm, ws, b):
    has_skip = xskip is not None

    def body(ag, s_r, *refs):
        if has_skip:
            xs_r, wm_r, ws_r, b_r, y_o, m_o, s_o = refs
        else:
            wm_r, ws_r, b_r, y_o, m_o, s_o = refs
        y = jnp.maximum(ag[0:_N, :] + ag[_N:2 * _N, :] + s_r[...], 0.0)
        if has_skip:
            y = y + xs_r[...]
        y_o[...] = y
        xn = _ln(y)
        m_o[...] = _dot(xn, wm_r[...])
        s_o[...] = _dot(xn, ws_r[...]) + b_r[...]

    nd = jax.ShapeDtypeStruct((_N, _D), jnp.float32)
    args = (aggp, s) + ((xskip,) if has_skip else ()) + (
        wm, ws, b.reshape(1, _D))
    return pl.pallas_call(body, out_shape=(nd, nd, nd))(*args)


def _final(aggp, s):
    def body(ag, s_r, out_o):
        y = jnp.maximum(ag[0:_N, :] + ag[_N:2 * _N, :] + s_r[...], 0.0)
        out_o[...] = _ln(y)

    return pl.pallas_call(
        body, out_shape=jax.ShapeDtypeStruct((_N, _D), jnp.float32))(aggp, s)


def kernel(feat, pos, edge_index, W0m, W0s, b0, R0a, r0a, R0b, r0b,
           Wmm, Wms, bm, Rma, rma, Rmb, rmb, W1m, W1s, b1, R1a, r1a,
           R1b, r1b):
    src = edge_index[0]
    dst = edge_index[1]
    px = pos[:, 0]
    py = pos[:, 1]
    pz = pos[:, 2]

    d2 = _edge_dist2(px, py, pz, src, dst)
    g0, gm, g1 = _gates(d2, R0a, r0a, R0b, r0b, Rma, rma, Rmb, rmb,
                        R1a, r1a, R1b, r1b)
    z = jnp.zeros((_N, _D), jnp.float32)

    m, s = _pre(feat, W0m, W0s, b0)
    aggp = _conv_edges(m, g0, src, dst, z)
    x, m, s = _combine(aggp, s, None, Wmm, Wms, bm)
    for i in range(3):
        aggp = _conv_edges(m, gm, src, dst, z)
        if i < 2:
            x, m, s = _combine(aggp, s, x, Wmm, Wms, bm)
        else:
            x, m, s = _combine(aggp, s, x, W1m, W1s, b1)
    aggp = _conv_edges(m, g1, src, dst, z)
    return _final(aggp, s)


# speedup vs baseline: 6.0829x; 1.1137x over previous
import functools

import jax
import jax.numpy as jnp
from jax import lax
from jax.experimental import pallas as pl
from jax.experimental.pallas import tpu as pltpu
from jax.experimental.pallas import tpu_sc as plsc

_N = 10000
_E = 320000
_D = 128
_H = 32
_NC = 2
_NS = 16
_NW = _NC * _NS
_L = 16
_W = 64
_NWIN = _E // _W
_WIN_BASE = _NWIN // _NW
_WIN_REM = _NWIN % _NW
_RPT = 624
_RTAIL = _N - _NS * _RPT


def _sc_mesh():
    return plsc.VectorSubcoreMesh(
        core_axis_name="c", subcore_axis_name="s",
        num_cores=_NC, num_subcores=_NS)


_SC_PARAMS = pltpu.CompilerParams(needs_layout_passes=False)


def _worker_id():
    return lax.axis_index("s") * _NC + lax.axis_index("c")


def _num_windows(wid):
    return _WIN_BASE + jnp.where(wid < _WIN_REM, 1, 0).astype(jnp.int32)


_W2 = 512
_NWIN2 = _E // _W2
_WIN2_BASE = _NWIN2 // _NW
_WIN2_REM = _NWIN2 % _NW
_TRIP2 = 20


def _edge_dist2(px, py, pz, src, dst):
    @functools.partial(
        pl.kernel,
        out_type=jax.ShapeDtypeStruct((_E,), jnp.float32),
        mesh=_sc_mesh(),
        compiler_params=_SC_PARAMS,
        scratch_types=(
            [pltpu.VMEM((_N,), jnp.float32)] * 3
            + [pltpu.VMEM((_W2,), jnp.int32)] * 4
            + [pltpu.VMEM((_W2,), jnp.float32)] * 2
            + [pltpu.SemaphoreType.DMA] * 4
        ),
    )
    def k(px_h, py_h, pz_h, src_h, dst_h, d2_h, px_v, py_v, pz_v,
          sv0, sv1, dv0, dv1, o0, o1, is0, is1, os0, os1):
        srcb = [sv0, sv1]
        dstb = [dv0, dv1]
        outb = [o0, o1]
        isem = [is0, is1]
        osem = [os0, os1]
        wid = _worker_id()
        n = _WIN2_BASE + jnp.where(wid < _WIN2_REM, 1, 0).astype(jnp.int32)

        def base_of(w):
            return (wid + _NW * jnp.minimum(w, n - 1)) * _W2

        def issue_idx(w, j):
            b = base_of(w)
            pltpu.async_copy(src_h.at[pl.ds(b, _W2)], srcb[j], isem[j])
            pltpu.async_copy(dst_h.at[pl.ds(b, _W2)], dstb[j], isem[j])

        def wait_idx(j):
            pltpu.make_async_copy(
                src_h.at[pl.ds(0, _W2)], srcb[j], isem[j]).wait()
            pltpu.make_async_copy(
                dst_h.at[pl.ds(0, _W2)], dstb[j], isem[j]).wait()

        pltpu.sync_copy(px_h, px_v)
        pltpu.sync_copy(py_h, py_v)
        pltpu.sync_copy(pz_h, pz_v)
        issue_idx(0, 0)

        @pl.loop(0, _TRIP2, step=2)
        def outer(i):
            for b in range(2):
                w = i + b
                p = b % 2
                wait_idx(p)
                issue_idx(w + 1, 1 - p)

                @pl.when(jnp.logical_and(w >= 2, w - 2 < n))
                def _store_done():
                    pltpu.make_async_copy(
                        outb[p], d2_h.at[pl.ds(0, _W2)], osem[p]).wait()

                @plsc.parallel_loop(0, _W2 // _L, unroll=2)
                def _blk(kk):
                    sl = pl.ds(kk * _L, _L)
                    sv = srcb[p][sl]
                    dv = dstb[p][sl]
                    dx = (plsc.load_gather(px_v, [sv])
                          - plsc.load_gather(px_v, [dv]))
                    dy = (plsc.load_gather(py_v, [sv])
                          - plsc.load_gather(py_v, [dv]))
                    dz = (plsc.load_gather(pz_v, [sv])
                          - plsc.load_gather(pz_v, [dv]))
                    outb[p][sl] = dx * dx + dy * dy + dz * dz

                @pl.when(w < n)
                def _store():
                    pltpu.async_copy(
                        outb[p], d2_h.at[pl.ds(base_of(w), _W2)], osem[p])

        wait_idx(_TRIP2 % 2)
        @pl.when(_TRIP2 - 2 < n)
        def _dr0():
            pltpu.make_async_copy(
                outb[0], d2_h.at[pl.ds(0, _W2)], osem[0]).wait()
        @pl.when(_TRIP2 - 1 < n)
        def _dr1():
            pltpu.make_async_copy(
                outb[1], d2_h.at[pl.ds(0, _W2)], osem[1]).wait()

    return k(px, py, pz, src, dst)


_TRIP = 160


def _conv_edges(m, gate, src, dst, zeros_nd):
    @functools.partial(
        pl.kernel,
        out_type=jax.ShapeDtypeStruct((_NC * _N, _D), jnp.float32),
        mesh=_sc_mesh(),
        compiler_params=_SC_PARAMS,
        scratch_types=(
            [pltpu.VMEM_SHARED((_N, _D), jnp.float32)]
            + [pltpu.VMEM((_W,), jnp.int32)] * 8
            + [pltpu.VMEM((_W, _D), jnp.float32)] * 4
            + [pltpu.SemaphoreType.DMA] * 10
        ),
    )
    def k(m_h, g_h, src_h, dst_h, z_h, out_h, agg_s,
          sb0, sb1, sb2, sb3, db0, db1, db2, db3, r0, r1, gb0, gb1,
          gs0, gs1, gts0, gts1, ss0, ss1, is0, is1, is2, is3):
        srcb = [sb0, sb1, sb2, sb3]
        dstb = [db0, db1, db2, db3]
        rows = [r0, r1]
        gateb = [gb0, gb1]
        gsem = [gs0, gs1]
        gtsem = [gts0, gts1]
        ssem = [ss0, ss1]
        isem = [is0, is1, is2, is3]

        cid = lax.axis_index("c")
        sid = lax.axis_index("s")
        wid = _worker_id()
        n = _num_windows(wid)

        def base_of(w):
            return (wid + _NW * jnp.minimum(w, n - 1)) * _W

        def issue_idx(w, j):
            b = base_of(w)
            pltpu.async_copy(src_h.at[pl.ds(b, _W)], srcb[j], isem[j])
            pltpu.async_copy(dst_h.at[pl.ds(b, _W)], dstb[j], isem[j])

        def wait_idx(j):
            pltpu.make_async_copy(
                src_h.at[pl.ds(0, _W)], srcb[j], isem[j]).wait()
            pltpu.make_async_copy(
                dst_h.at[pl.ds(0, _W)], dstb[j], isem[j]).wait()

        def issue_data(w, j, p):
            pltpu.async_copy(m_h.at[srcb[j]], rows[p], gsem[p])
            pltpu.async_copy(g_h.at[pl.ds(base_of(w), _W)], gateb[p],
                             gtsem[p])

        def wait_data(j, p):
            pltpu.make_async_copy(m_h.at[srcb[j]], rows[p], gsem[p]).wait()
            pltpu.make_async_copy(
                g_h.at[pl.ds(0, _W)], gateb[p], gtsem[p]).wait()

        def issue_scatter(j, p):
            pltpu.async_copy(rows[p], agg_s.at[dstb[j]], ssem[p], add=True)

        def wait_scatter(j, p):
            pltpu.make_async_copy(rows[p], agg_s.at[dstb[j]], ssem[p]).wait()

        pltpu.sync_copy(z_h.at[pl.ds(sid * _RPT, _RPT)],
                        agg_s.at[pl.ds(sid * _RPT, _RPT)])
        @pl.when(sid == 0)
        def _zero_tail():
            pltpu.sync_copy(z_h.at[pl.ds(_NS * _RPT, _RTAIL)],
                            agg_s.at[pl.ds(_NS * _RPT, _RTAIL)])
        plsc.subcore_barrier()

        pltpu.sync_copy(src_h.at[pl.ds(base_of(0), _W)], srcb[0])
        pltpu.sync_copy(dst_h.at[pl.ds(base_of(0), _W)], dstb[0])
        issue_data(0, 0, 0)
        issue_idx(1, 1)

        @pl.loop(0, _TRIP, step=4)
        def outer(i):
            for b in range(4):
                w = i + b
                p = b % 2
                q = (b + 1) % 2

                @pl.when(jnp.logical_and(w >= 1, w - 1 < n))
                def _done_prev():
                    wait_scatter((b + 3) % 4, q)

                wait_idx((b + 1) % 4)
                issue_data(w + 1, (b + 1) % 4, q)
                issue_idx(w + 2, (b + 2) % 4)
                wait_data(b, p)

                @plsc.parallel_loop(0, _W, unroll=2)
                def _mul(r):
                    for jj in range(_D // _L):
                        sl = pl.ds(jj * _L, _L)
                        rows[p][r, sl] = rows[p][r, sl] * gateb[p][r, sl]

                @pl.when(w < n)
                def _scatter():
                    issue_scatter(b, p)

        wait_data(0, 0)
        wait_idx(1)
        plsc.subcore_barrier()
        pltpu.sync_copy(
            agg_s.at[pl.ds(sid * _RPT, _RPT)],
            out_h.at[pl.ds(cid * _N + sid * _RPT, _RPT)])
        @pl.when(sid == 0)
        def _out_tail():
            pltpu.sync_copy(
                agg_s.at[pl.ds(_NS * _RPT, _RTAIL)],
                out_h.at[pl.ds(cid * _N + _NS * _RPT, _RTAIL)])

    return k(m, gate, src, dst, zeros_nd)


def _ln(x):
    x = x - jnp.mean(x, axis=-1, keepdims=True)
    return x / jnp.sqrt(jnp.mean(x * x, axis=-1, keepdims=True) + 1e-5)


def _dot(a, b):
    return jnp.dot(a, b, preferred_element_type=jnp.float32)


_EB = 512
_EGRID = _E // _EB


def _gates(d2, R0a, r0a, R0b, r0b, Rma, rma, Rmb, rmb, R1a, r1a, R1b, r1b):
    def body(d2_ref, a0, c0, b0_, d0, am, cm, bm_, dm, a1, c1, b1_, d1,
             g0_o, gm_o, g1_o):
        dcol = jnp.sqrt(d2_ref[...] + 1e-12).reshape(_EB, 1)

        def one(Ra, ra, Rb, rb):
            h = jnp.maximum(dcol * Ra[...] + ra[...], 0.0)
            return _dot(h, Rb[...]) + rb[...]

        g0_o[...] = one(a0, c0, b0_, d0)
        gm_o[...] = one(am, cm, bm_, dm)
        g1_o[...] = one(a1, c1, b1_, d1)

    w = [R0a, r0a.reshape(1, _H), R0b, r0b.reshape(1, _D),
         Rma, rma.reshape(1, _H), Rmb, rmb.reshape(1, _D),
         R1a, r1a.reshape(1, _H), R1b, r1b.reshape(1, _D)]
    wspecs = [pl.BlockSpec(x.shape, lambda i: (0,) * x.ndim) for x in w]
    gs = jax.ShapeDtypeStruct((_E, _D), jnp.float32)
    return pl.pallas_call(
        body,
        grid=(_EGRID,),
        in_specs=[pl.BlockSpec((_EB,), lambda i: (i,))] + wspecs,
        out_specs=[pl.BlockSpec((_EB, _D), lambda i: (i, 0))] * 3,
        out_shape=(gs, gs, gs),
    )(d2, *w)


def _pre(feat, W0m, W0s, b0):
    def body(f, wm, ws, b, m_o, s_o):
        xn = _ln(f[...])
        m_o[...] = _dot(xn, wm[...])
        s_o[...] = _dot(xn, ws[...]) + b[...]

    nd = jax.ShapeDtypeStruct((_N, _D), jnp.float32)
    return pl.pallas_call(body, out_shape=(nd, nd))(
        feat, W0m, W0s, b0.reshape(1, _D))


def _combine(aggp, s, xskip, wm, ws, b):
    has_skip = xskip is not None

    def body(ag, s_r, *refs):
        if has_skip:
            xs_r, wm_r, ws_r, b_r, y_o, m_o, s_o = refs
        else:
            wm_r, ws_r, b_r, y_o, m_o, s_o = refs
        y = jnp.maximum(ag[0:_N, :] + ag[_N:2 * _N, :] + s_r[...], 0.0)
        if has_skip:
            y = y + xs_r[...]
        y_o[...] = y
        xn = _ln(y)
        m_o[...] = _dot(xn, wm_r[...])
        s_o[...] = _dot(xn, ws_r[...]) + b_r[...]

    nd = jax.ShapeDtypeStruct((_N, _D), jnp.float32)
    args = (aggp, s) + ((xskip,) if has_skip else ()) + (
        wm, ws, b.reshape(1, _D))
    return pl.pallas_call(body, out_shape=(nd, nd, nd))(*args)


def _final(aggp, s):
    def body(ag, s_r, out_o):
        y = jnp.maximum(ag[0:_N, :] + ag[_N:2 * _N, :] + s_r[...], 0.0)
        out_o[...] = _ln(y)

    return pl.pallas_call(
        body, out_shape=jax.ShapeDtypeStruct((_N, _D), jnp.float32))(aggp, s)


def kernel(feat, pos, edge_index, W0m, W0s, b0, R0a, r0a, R0b, r0b,
           Wmm, Wms, bm, Rma, rma, Rmb, rmb, W1m, W1s, b1, R1a, r1a,
           R1b, r1b):
    src = edge_index[0]
    dst = edge_index[1]
    px = pos[:, 0]
    py = pos[:, 1]
    pz = pos[:, 2]

    d2 = _edge_dist2(px, py, pz, src, dst)
    g0, gm, g1 = _gates(d2, R0a, r0a, R0b, r0b, Rma, rma, Rmb, rmb,
                        R1a, r1a, R1b, r1b)
    z = jnp.zeros((_N, _D), jnp.float32)

    m, s = _pre(feat, W0m, W0s, b0)
    aggp = _conv_edges(m, g0, src, dst, z)
    x, m, s = _combine(aggp, s, None, Wmm, Wms, bm)
    for i in range(3):
        aggp = _conv_edges(m, gm, src, dst, z)
        if i < 2:
            x, m, s = _combine(aggp, s, x, Wmm, Wms, bm)
        else:
            x, m, s = _combine(aggp, s, x, W1m, W1s, b1)
    aggp = _conv_edges(m, g1, src, dst, z)
    return _final(aggp, s)


# speedup vs baseline: 7.2454x; 1.1911x over previous
import functools

import jax
import jax.numpy as jnp
from jax import lax
from jax.experimental import pallas as pl
from jax.experimental.pallas import tpu as pltpu
from jax.experimental.pallas import tpu_sc as plsc

_N = 10000
_E = 320000
_D = 128
_H = 32
_NC = 2
_NS = 16
_NW = _NC * _NS
_L = 16
_W = 64
_NWIN = _E // _W
_WIN_BASE = _NWIN // _NW
_WIN_REM = _NWIN % _NW
_RPT = 624
_RTAIL = _N - _NS * _RPT


def _sc_mesh():
    return plsc.VectorSubcoreMesh(
        core_axis_name="c", subcore_axis_name="s",
        num_cores=_NC, num_subcores=_NS)


_SC_PARAMS = pltpu.CompilerParams(needs_layout_passes=False)


def _worker_id():
    return lax.axis_index("s") * _NC + lax.axis_index("c")


def _num_windows(wid):
    return _WIN_BASE + jnp.where(wid < _WIN_REM, 1, 0).astype(jnp.int32)


_W2 = 512
_NWIN2 = _E // _W2
_WIN2_BASE = _NWIN2 // _NW
_WIN2_REM = _NWIN2 % _NW
_TRIP2 = 20


def _edge_dist2(px, py, pz, src, dst):
    @functools.partial(
        pl.kernel,
        out_type=jax.ShapeDtypeStruct((_E,), jnp.float32),
        mesh=_sc_mesh(),
        compiler_params=_SC_PARAMS,
        scratch_types=(
            [pltpu.VMEM((_N,), jnp.float32)] * 3
            + [pltpu.VMEM((_W2,), jnp.int32)] * 4
            + [pltpu.VMEM((_W2,), jnp.float32)] * 2
            + [pltpu.SemaphoreType.DMA] * 4
        ),
    )
    def k(px_h, py_h, pz_h, src_h, dst_h, d2_h, px_v, py_v, pz_v,
          sv0, sv1, dv0, dv1, o0, o1, is0, is1, os0, os1):
        srcb = [sv0, sv1]
        dstb = [dv0, dv1]
        outb = [o0, o1]
        isem = [is0, is1]
        osem = [os0, os1]
        wid = _worker_id()
        n = _WIN2_BASE + jnp.where(wid < _WIN2_REM, 1, 0).astype(jnp.int32)

        def base_of(w):
            return (wid + _NW * jnp.minimum(w, n - 1)) * _W2

        def issue_idx(w, j):
            b = base_of(w)
            pltpu.async_copy(src_h.at[pl.ds(b, _W2)], srcb[j], isem[j])
            pltpu.async_copy(dst_h.at[pl.ds(b, _W2)], dstb[j], isem[j])

        def wait_idx(j):
            pltpu.make_async_copy(
                src_h.at[pl.ds(0, _W2)], srcb[j], isem[j]).wait()
            pltpu.make_async_copy(
                dst_h.at[pl.ds(0, _W2)], dstb[j], isem[j]).wait()

        pltpu.sync_copy(px_h, px_v)
        pltpu.sync_copy(py_h, py_v)
        pltpu.sync_copy(pz_h, pz_v)
        issue_idx(0, 0)

        @pl.loop(0, _TRIP2, step=2)
        def outer(i):
            for b in range(2):
                w = i + b
                p = b % 2
                wait_idx(p)
                issue_idx(w + 1, 1 - p)

                @pl.when(jnp.logical_and(w >= 2, w - 2 < n))
                def _store_done():
                    pltpu.make_async_copy(
                        outb[p], d2_h.at[pl.ds(0, _W2)], osem[p]).wait()

                @plsc.parallel_loop(0, _W2 // _L, unroll=2)
                def _blk(kk):
                    sl = pl.ds(kk * _L, _L)
                    sv = srcb[p][sl]
                    dv = dstb[p][sl]
                    dx = (plsc.load_gather(px_v, [sv])
                          - plsc.load_gather(px_v, [dv]))
                    dy = (plsc.load_gather(py_v, [sv])
                          - plsc.load_gather(py_v, [dv]))
                    dz = (plsc.load_gather(pz_v, [sv])
                          - plsc.load_gather(pz_v, [dv]))
                    outb[p][sl] = dx * dx + dy * dy + dz * dz

                @pl.when(w < n)
                def _store():
                    pltpu.async_copy(
                        outb[p], d2_h.at[pl.ds(base_of(w), _W2)], osem[p])

        wait_idx(_TRIP2 % 2)
        @pl.when(_TRIP2 - 2 < n)
        def _dr0():
            pltpu.make_async_copy(
                outb[0], d2_h.at[pl.ds(0, _W2)], osem[0]).wait()
        @pl.when(_TRIP2 - 1 < n)
        def _dr1():
            pltpu.make_async_copy(
                outb[1], d2_h.at[pl.ds(0, _W2)], osem[1]).wait()

    return k(px, py, pz, src, dst)


_TRIP = 160


def _conv_edges(m, gate, src, dst, zeros_nd):
    @functools.partial(
        pl.kernel,
        out_type=jax.ShapeDtypeStruct((_NC * _N, _D), jnp.float32),
        mesh=_sc_mesh(),
        compiler_params=_SC_PARAMS,
        scratch_types=(
            [pltpu.VMEM_SHARED((_N, _D), jnp.float32)]
            + [pltpu.VMEM((_W,), jnp.int32)] * 8
            + [pltpu.VMEM((_W, _D), jnp.float32)] * 4
            + [pltpu.SemaphoreType.DMA] * 10
        ),
    )
    def k(m_h, g_h, src_h, dst_h, z_h, out_h, agg_s,
          sb0, sb1, sb2, sb3, db0, db1, db2, db3, r0, r1, gb0, gb1,
          gs0, gs1, gts0, gts1, ss0, ss1, is0, is1, is2, is3):
        srcb = [sb0, sb1, sb2, sb3]
        dstb = [db0, db1, db2, db3]
        rows = [r0, r1]
        gateb = [gb0, gb1]
        gsem = [gs0, gs1]
        gtsem = [gts0, gts1]
        ssem = [ss0, ss1]
        isem = [is0, is1, is2, is3]

        cid = lax.axis_index("c")
        sid = lax.axis_index("s")
        wid = _worker_id()
        n = _num_windows(wid)

        def base_of(w):
            return (wid + _NW * jnp.minimum(w, n - 1)) * _W

        def issue_idx(w, j):
            b = base_of(w)
            pltpu.async_copy(src_h.at[pl.ds(b, _W)], srcb[j], isem[j])
            pltpu.async_copy(dst_h.at[pl.ds(b, _W)], dstb[j], isem[j])

        def wait_idx(j):
            pltpu.make_async_copy(
                src_h.at[pl.ds(0, _W)], srcb[j], isem[j]).wait()
            pltpu.make_async_copy(
                dst_h.at[pl.ds(0, _W)], dstb[j], isem[j]).wait()

        def issue_data(w, j, p):
            pltpu.async_copy(m_h.at[srcb[j]], rows[p], gsem[p])
            pltpu.async_copy(g_h.at[pl.ds(base_of(w), _W)], gateb[p],
                             gtsem[p])

        def wait_data(j, p):
            pltpu.make_async_copy(m_h.at[srcb[j]], rows[p], gsem[p]).wait()
            pltpu.make_async_copy(
                g_h.at[pl.ds(0, _W)], gateb[p], gtsem[p]).wait()

        def issue_scatter(j, p):
            pltpu.async_copy(rows[p], agg_s.at[dstb[j]], ssem[p], add=True)

        def wait_scatter(j, p):
            pltpu.make_async_copy(rows[p], agg_s.at[dstb[j]], ssem[p]).wait()

        pltpu.sync_copy(z_h.at[pl.ds(sid * _RPT, _RPT)],
                        agg_s.at[pl.ds(sid * _RPT, _RPT)])
        @pl.when(sid == 0)
        def _zero_tail():
            pltpu.sync_copy(z_h.at[pl.ds(_NS * _RPT, _RTAIL)],
                            agg_s.at[pl.ds(_NS * _RPT, _RTAIL)])
        plsc.subcore_barrier()

        pltpu.sync_copy(src_h.at[pl.ds(base_of(0), _W)], srcb[0])
        pltpu.sync_copy(dst_h.at[pl.ds(base_of(0), _W)], dstb[0])
        issue_data(0, 0, 0)
        issue_idx(1, 1)

        @pl.loop(0, _TRIP, step=4)
        def outer(i):
            for b in range(4):
                w = i + b
                p = b % 2
                q = (b + 1) % 2

                @pl.when(jnp.logical_and(w >= 1, w - 1 < n))
                def _done_prev():
                    wait_scatter((b + 3) % 4, q)

                wait_idx((b + 1) % 4)
                issue_data(w + 1, (b + 1) % 4, q)
                issue_idx(w + 2, (b + 2) % 4)
                wait_data(b, p)

                @plsc.parallel_loop(0, _W, unroll=2)
                def _mul(r):
                    for jj in range(_D // _L):
                        sl = pl.ds(jj * _L, _L)
                        rows[p][r, sl] = rows[p][r, sl] * gateb[p][r, sl]

                @pl.when(w < n)
                def _scatter():
                    issue_scatter(b, p)

        wait_data(0, 0)
        wait_idx(1)
        plsc.subcore_barrier()
        pltpu.sync_copy(
            agg_s.at[pl.ds(sid * _RPT, _RPT)],
            out_h.at[pl.ds(cid * _N + sid * _RPT, _RPT)])
        @pl.when(sid == 0)
        def _out_tail():
            pltpu.sync_copy(
                agg_s.at[pl.ds(_NS * _RPT, _RTAIL)],
                out_h.at[pl.ds(cid * _N + _NS * _RPT, _RTAIL)])

    return k(m, gate, src, dst, zeros_nd)


def _ln(x):
    x = x - jnp.mean(x, axis=-1, keepdims=True)
    return x / jnp.sqrt(jnp.mean(x * x, axis=-1, keepdims=True) + 1e-5)


def _dot(a, b):
    return jnp.dot(a, b, preferred_element_type=jnp.float32)


_EB = 8192
_EPAD = _EB * 40
_EGRID = _EPAD // _EB


def _gates(d2, R0a, r0a, R0b, r0b, Rma, rma, Rmb, rmb, R1a, r1a, R1b, r1b):
    def body(d2_ref, a0, c0, b0_, d0, am, cm, bm_, dm, a1, c1, b1_, d1,
             g0_o, gm_o, g1_o):
        dcol = jnp.sqrt(d2_ref[...] + 1e-12).reshape(_EB, 1)

        def one(Ra, ra, Rb, rb):
            h = jnp.maximum(dcol * Ra[...] + ra[...], 0.0)
            return _dot(h, Rb[...]) + rb[...]

        g0_o[...] = one(a0, c0, b0_, d0)
        gm_o[...] = one(am, cm, bm_, dm)
        g1_o[...] = one(a1, c1, b1_, d1)

    w = [R0a, r0a.reshape(1, _H), R0b, r0b.reshape(1, _D),
         Rma, rma.reshape(1, _H), Rmb, rmb.reshape(1, _D),
         R1a, r1a.reshape(1, _H), R1b, r1b.reshape(1, _D)]
    wspecs = [pl.BlockSpec(x.shape, lambda i: (0,) * x.ndim) for x in w]
    gs = jax.ShapeDtypeStruct((_EPAD, _D), jnp.float32)
    d2p = jnp.concatenate([d2, jnp.zeros((_EPAD - _E,), jnp.float32)])
    return pl.pallas_call(
        body,
        grid=(_EGRID,),
        in_specs=[pl.BlockSpec((_EB,), lambda i: (i,))] + wspecs,
        out_specs=[pl.BlockSpec((_EB, _D), lambda i: (i, 0))] * 3,
        out_shape=(gs, gs, gs),
    )(d2p, *w)


def _pre(feat, W0m, W0s, b0):
    def body(f, wm, ws, b, m_o, s_o):
        xn = _ln(f[...])
        m_o[...] = _dot(xn, wm[...])
        s_o[...] = _dot(xn, ws[...]) + b[...]

    nd = jax.ShapeDtypeStruct((_N, _D), jnp.float32)
    return pl.pallas_call(body, out_shape=(nd, nd))(
        feat, W0m, W0s, b0.reshape(1, _D))


def _combine(aggp, s, xskip, wm, ws, b):
    has_skip = xskip is not None

    def body(ag, s_r, *refs):
        if has_skip:
            xs_r, wm_r, ws_r, b_r, y_o, m_o, s_o = refs
        else:
            wm_r, ws_r, b_r, y_o, m_o, s_o = refs
        y = jnp.maximum(ag[0:_N, :] + ag[_N:2 * _N, :] + s_r[...], 0.0)
        if has_skip:
            y = y + xs_r[...]
        y_o[...] = y
        xn = _ln(y)
        m_o[...] = _dot(xn, wm_r[...])
        s_o[...] = _dot(xn, ws_r[...]) + b_r[...]

    nd = jax.ShapeDtypeStruct((_N, _D), jnp.float32)
    args = (aggp, s) + ((xskip,) if has_skip else ()) + (
        wm, ws, b.reshape(1, _D))
    return pl.pallas_call(body, out_shape=(nd, nd, nd))(*args)


def _final(aggp, s):
    def body(ag, s_r, out_o):
        y = jnp.maximum(ag[0:_N, :] + ag[_N:2 * _N, :] + s_r[...], 0.0)
        out_o[...] = _ln(y)

    return pl.pallas_call(
        body, out_shape=jax.ShapeDtypeStruct((_N, _D), jnp.float32))(aggp, s)


def kernel(feat, pos, edge_index, W0m, W0s, b0, R0a, r0a, R0b, r0b,
           Wmm, Wms, bm, Rma, rma, Rmb, rmb, W1m, W1s, b1, R1a, r1a,
           R1b, r1b):
    src = edge_index[0]
    dst = edge_index[1]
    px = pos[:, 0]
    py = pos[:, 1]
    pz = pos[:, 2]

    d2 = _edge_dist2(px, py, pz, src, dst)
    g0, gm, g1 = _gates(d2, R0a, r0a, R0b, r0b, Rma, rma, Rmb, rmb,
                        R1a, r1a, R1b, r1b)
    z = jnp.zeros((_N, _D), jnp.float32)

    m, s = _pre(feat, W0m, W0s, b0)
    aggp = _conv_edges(m, g0, src, dst, z)
    x, m, s = _combine(aggp, s, None, Wmm, Wms, bm)
    for i in range(3):
        aggp = _conv_edges(m, gm, src, dst, z)
        if i < 2:
            x, m, s = _combine(aggp, s, x, Wmm, Wms, bm)
        else:
            x, m, s = _combine(aggp, s, x, W1m, W1s, b1)
    aggp = _conv_edges(m, g1, src, dst, z)
    return _final(aggp, s)


# speedup vs baseline: 7.5668x; 1.0443x over previous
import functools

import jax
import jax.numpy as jnp
from jax import lax
from jax.experimental import pallas as pl
from jax.experimental.pallas import tpu as pltpu
from jax.experimental.pallas import tpu_sc as plsc

_N = 10000
_E = 320000
_D = 128
_H = 32
_NC = 2
_NS = 16
_NW = _NC * _NS
_L = 16
_EPW = _E // _NW
_W = 80
_NWIN = _EPW // _W
_RPT = 624
_RTAIL = _N - _NS * _RPT


def _sc_mesh():
    return plsc.VectorSubcoreMesh(
        core_axis_name="c", subcore_axis_name="s",
        num_cores=_NC, num_subcores=_NS)


_SC_PARAMS = pltpu.CompilerParams(needs_layout_passes=False)


def _worker_id():
    return lax.axis_index("s") * _NC + lax.axis_index("c")


_W2 = 1000
_NWIN2 = _EPW // _W2


def _edge_dist2(px, py, pz, src, dst):
    @functools.partial(
        pl.kernel,
        out_type=jax.ShapeDtypeStruct((_E,), jnp.float32),
        mesh=_sc_mesh(),
        compiler_params=_SC_PARAMS,
        scratch_types=(
            [pltpu.VMEM((_N,), jnp.float32)] * 3
            + [pltpu.VMEM((_W2,), jnp.int32)] * 4
            + [pltpu.VMEM((_W2,), jnp.float32)] * 2
            + [pltpu.SemaphoreType.DMA] * 4
        ),
    )
    def k(px_h, py_h, pz_h, src_h, dst_h, d2_h, px_v, py_v, pz_v,
          sv0, sv1, dv0, dv1, o0, o1, is0, is1, os0, os1):
        srcb = [sv0, sv1]
        dstb = [dv0, dv1]
        outb = [o0, o1]
        isem = [is0, is1]
        osem = [os0, os1]
        wid = _worker_id()

        def base_of(w):
            return wid * _EPW + w * _W2

        def issue_idx(w, j):
            b = base_of(w)
            pltpu.async_copy(src_h.at[pl.ds(b, _W2)], srcb[j], isem[j])
            pltpu.async_copy(dst_h.at[pl.ds(b, _W2)], dstb[j], isem[j])

        def wait_idx(j):
            pltpu.make_async_copy(
                src_h.at[pl.ds(0, _W2)], srcb[j], isem[j]).wait()
            pltpu.make_async_copy(
                dst_h.at[pl.ds(0, _W2)], dstb[j], isem[j]).wait()

        def wait_store(j):
            pltpu.make_async_copy(
                outb[j], d2_h.at[pl.ds(0, _W2)], osem[j]).wait()

        pltpu.sync_copy(px_h, px_v)
        pltpu.sync_copy(py_h, py_v)
        pltpu.sync_copy(pz_h, pz_v)
        issue_idx(0, 0)

        @pl.loop(0, _NWIN2, step=2)
        def outer(i):
            for b in range(2):
                w = i + b
                wait_idx(b)

                if b == 0:
                    issue_idx(w + 1, 1)
                else:
                    @pl.when(i < _NWIN2 - 2)
                    def _pf():
                        issue_idx(w + 1, 0)

                @pl.when(w >= 2)
                def _store_done():
                    wait_store(b)

                @plsc.parallel_loop(0, _W2 // _L, unroll=2)
                def _blk(kk):
                    sl = pl.ds(kk * _L, _L)
                    sv = srcb[b][sl]
                    dv = dstb[b][sl]
                    dx = (plsc.load_gather(px_v, [sv])
                          - plsc.load_gather(px_v, [dv]))
                    dy = (plsc.load_gather(py_v, [sv])
                          - plsc.load_gather(py_v, [dv]))
                    dz = (plsc.load_gather(pz_v, [sv])
                          - plsc.load_gather(pz_v, [dv]))
                    outb[b][sl] = dx * dx + dy * dy + dz * dz

                pltpu.async_copy(
                    outb[b], d2_h.at[pl.ds(base_of(w), _W2)], osem[b])

        wait_store(0)
        wait_store(1)

    return k(px, py, pz, src, dst)


def _conv_edges(m, gate, src, dst, zeros_nd):
    @functools.partial(
        pl.kernel,
        out_type=jax.ShapeDtypeStruct((_NC * _N, _D), jnp.float32),
        mesh=_sc_mesh(),
        compiler_params=_SC_PARAMS,
        scratch_types=(
            [pltpu.VMEM_SHARED((_N, _D), jnp.float32)]
            + [pltpu.VMEM((_W,), jnp.int32)] * 8
            + [pltpu.VMEM((_W, _D), jnp.float32)] * 4
            + [pltpu.SemaphoreType.DMA] * 10
        ),
    )
    def k(m_h, g_h, src_h, dst_h, z_h, out_h, agg_s,
          sb0, sb1, sb2, sb3, db0, db1, db2, db3, r0, r1, gb0, gb1,
          gs0, gs1, gts0, gts1, ss0, ss1, is0, is1, is2, is3):
        srcb = [sb0, sb1, sb2, sb3]
        dstb = [db0, db1, db2, db3]
        rows = [r0, r1]
        gateb = [gb0, gb1]
        gsem = [gs0, gs1]
        gtsem = [gts0, gts1]
        ssem = [ss0, ss1]
        isem = [is0, is1, is2, is3]

        cid = lax.axis_index("c")
        sid = lax.axis_index("s")
        wid = _worker_id()

        def base_of(w):
            return wid * _EPW + w * _W

        def issue_idx(w, j):
            b = base_of(w)
            pltpu.async_copy(src_h.at[pl.ds(b, _W)], srcb[j], isem[j])
            pltpu.async_copy(dst_h.at[pl.ds(b, _W)], dstb[j], isem[j])

        def wait_idx(j):
            pltpu.make_async_copy(
                src_h.at[pl.ds(0, _W)], srcb[j], isem[j]).wait()
            pltpu.make_async_copy(
                dst_h.at[pl.ds(0, _W)], dstb[j], isem[j]).wait()

        def issue_data(w, j, p):
            pltpu.async_copy(m_h.at[srcb[j]], rows[p], gsem[p])
            pltpu.async_copy(g_h.at[pl.ds(base_of(w), _W)], gateb[p],
                             gtsem[p])

        def wait_data(j, p):
            pltpu.make_async_copy(m_h.at[srcb[j]], rows[p], gsem[p]).wait()
            pltpu.make_async_copy(
                g_h.at[pl.ds(0, _W)], gateb[p], gtsem[p]).wait()

        def issue_scatter(j, p):
            pltpu.async_copy(rows[p], agg_s.at[dstb[j]], ssem[p], add=True)

        def wait_scatter(j, p):
            pltpu.make_async_copy(rows[p], agg_s.at[dstb[j]], ssem[p]).wait()

        pltpu.sync_copy(z_h.at[pl.ds(sid * _RPT, _RPT)],
                        agg_s.at[pl.ds(sid * _RPT, _RPT)])
        @pl.when(sid == 0)
        def _zero_tail():
            pltpu.sync_copy(z_h.at[pl.ds(_NS * _RPT, _RTAIL)],
                            agg_s.at[pl.ds(_NS * _RPT, _RTAIL)])
        plsc.subcore_barrier()

        def mul_window(p):
            @plsc.parallel_loop(0, _W, unroll=2)
            def _mul(r):
                for jj in range(_D // _L):
                    sl = pl.ds(jj * _L, _L)
                    rows[p][r, sl] = rows[p][r, sl] * gateb[p][r, sl]

        pltpu.sync_copy(src_h.at[pl.ds(base_of(0), _W)], srcb[0])
        pltpu.sync_copy(dst_h.at[pl.ds(base_of(0), _W)], dstb[0])
        issue_data(0, 0, 0)
        issue_idx(1, 1)

        @pl.loop(0, _NWIN - 1, step=4)
        def outer(i):
            for b in range(4):
                w = i + b
                p = b % 2
                q = (b + 1) % 2

                if b == 0:
                    @pl.when(i > 0)
                    def _done_prev():
                        wait_scatter(3, q)
                else:
                    wait_scatter(b - 1, q)
                wait_idx((b + 1) % 4)
                issue_data(w + 1, (b + 1) % 4, q)

                @pl.when(w + 2 < _NWIN)
                def _pf_idx():
                    issue_idx(w + 2, (b + 2) % 4)

                wait_data(b, p)
                mul_window(p)
                issue_scatter(b, p)

        wait_scatter(3, 1)
        wait_data(0, 0)
        mul_window(0)
        issue_scatter(0, 0)
        wait_scatter(0, 0)
        plsc.subcore_barrier()
        pltpu.sync_copy(
            agg_s.at[pl.ds(sid * _RPT, _RPT)],
            out_h.at[pl.ds(cid * _N + sid * _RPT, _RPT)])
        @pl.when(sid == 0)
        def _out_tail():
            pltpu.sync_copy(
                agg_s.at[pl.ds(_NS * _RPT, _RTAIL)],
                out_h.at[pl.ds(cid * _N + _NS * _RPT, _RTAIL)])

    return k(m, gate, src, dst, zeros_nd)


def _ln(x):
    x = x - jnp.mean(x, axis=-1, keepdims=True)
    return x / jnp.sqrt(jnp.mean(x * x, axis=-1, keepdims=True) + 1e-5)


def _dot(a, b):
    return jnp.dot(a, b, preferred_element_type=jnp.float32)


_EB = 8192
_EPAD = _EB * 40
_EGRID = _EPAD // _EB


def _gates(d2, R0a, r0a, R0b, r0b, Rma, rma, Rmb, rmb, R1a, r1a, R1b, r1b):
    def body(d2_ref, a0, c0, b0_, d0, am, cm, bm_, dm, a1, c1, b1_, d1,
             g0_o, gm_o, g1_o):
        dcol = jnp.sqrt(d2_ref[...] + 1e-12).reshape(_EB, 1)

        def one(Ra, ra, Rb, rb):
            h = jnp.maximum(dcol * Ra[...] + ra[...], 0.0)
            return _dot(h, Rb[...]) + rb[...]

        g0_o[...] = one(a0, c0, b0_, d0)
        gm_o[...] = one(am, cm, bm_, dm)
        g1_o[...] = one(a1, c1, b1_, d1)

    w = [R0a, r0a.reshape(1, _H), R0b, r0b.reshape(1, _D),
         Rma, rma.reshape(1, _H), Rmb, rmb.reshape(1, _D),
         R1a, r1a.reshape(1, _H), R1b, r1b.reshape(1, _D)]
    wspecs = [pl.BlockSpec(x.shape, lambda i: (0,) * x.ndim) for x in w]
    gs = jax.ShapeDtypeStruct((_EPAD, _D), jnp.float32)
    d2p = jnp.concatenate([d2, jnp.zeros((_EPAD - _E,), jnp.float32)])
    return pl.pallas_call(
        body,
        grid=(_EGRID,),
        in_specs=[pl.BlockSpec((_EB,), lambda i: (i,))] + wspecs,
        out_specs=[pl.BlockSpec((_EB, _D), lambda i: (i, 0))] * 3,
        out_shape=(gs, gs, gs),
    )(d2p, *w)


def _pre(feat, W0m, W0s, b0):
    def body(f, wm, ws, b, m_o, s_o):
        xn = _ln(f[...])
        m_o[...] = _dot(xn, wm[...])
        s_o[...] = _dot(xn, ws[...]) + b[...]

    nd = jax.ShapeDtypeStruct((_N, _D), jnp.float32)
    return pl.pallas_call(body, out_shape=(nd, nd))(
        feat, W0m, W0s, b0.reshape(1, _D))


def _combine(aggp, s, xskip, wm, ws, b):
    has_skip = xskip is not None

    def body(ag, s_r, *refs):
        if has_skip:
            xs_r, wm_r, ws_r, b_r, y_o, m_o, s_o = refs
        else:
            wm_r, ws_r, b_r, y_o, m_o, s_o = refs
        y = jnp.maximum(ag[0:_N, :] + ag[_N:2 * _N, :] + s_r[...], 0.0)
        if has_skip:
            y = y + xs_r[...]
        y_o[...] = y
        xn = _ln(y)
        m_o[...] = _dot(xn, wm_r[...])
        s_o[...] = _dot(xn, ws_r[...]) + b_r[...]

    nd = jax.ShapeDtypeStruct((_N, _D), jnp.float32)
    args = (aggp, s) + ((xskip,) if has_skip else ()) + (
        wm, ws, b.reshape(1, _D))
    return pl.pallas_call(body, out_shape=(nd, nd, nd))(*args)


def _final(aggp, s):
    def body(ag, s_r, out_o):
        y = jnp.maximum(ag[0:_N, :] + ag[_N:2 * _N, :] + s_r[...], 0.0)
        out_o[...] = _ln(y)

    return pl.pallas_call(
        body, out_shape=jax.ShapeDtypeStruct((_N, _D), jnp.float32))(aggp, s)


def kernel(feat, pos, edge_index, W0m, W0s, b0, R0a, r0a, R0b, r0b,
           Wmm, Wms, bm, Rma, rma, Rmb, rmb, W1m, W1s, b1, R1a, r1a,
           R1b, r1b):
    src = edge_index[0]
    dst = edge_index[1]
    px = pos[:, 0]
    py = pos[:, 1]
    pz = pos[:, 2]

    d2 = _edge_dist2(px, py, pz, src, dst)
    g0, gm, g1 = _gates(d2, R0a, r0a, R0b, r0b, Rma, rma, Rmb, rmb,
                        R1a, r1a, R1b, r1b)
    z = jnp.zeros((_N, _D), jnp.float32)

    m, s = _pre(feat, W0m, W0s, b0)
    aggp = _conv_edges(m, g0, src, dst, z)
    x, m, s = _combine(aggp, s, None, Wmm, Wms, bm)
    for i in range(3):
        aggp = _conv_edges(m, gm, src, dst, z)
        if i < 2:
            x, m, s = _combine(aggp, s, x, Wmm, Wms, bm)
        else:
            x, m, s = _combine(aggp, s, x, W1m, W1s, b1)
    aggp = _conv_edges(m, g1, src, dst, z)
    return _final(aggp, s)
